# baseline (speedup 1.0000x reference)
import functools

import jax
import jax.numpy as jnp
from jax import lax
from jax.experimental import pallas as pl
from jax.experimental.pallas import tpu as pltpu
from jax.experimental.pallas import tpu_sc as plsc

N_NODES = 10000
N_EDGES = 320000
IN_D = 128
HID = 64

NC = 2
NS = 16
NW = NC * NS

NP = 10240
RPT = NP // NS
EPT = N_EDGES // NW
K = 80
NCHUNK = EPT // K
NBUF = 5
NGRP = NCHUNK // NBUF

_mesh = plsc.VectorSubcoreMesh(core_axis_name="c", subcore_axis_name="s")
_sc_params = pltpu.CompilerParams(use_tc_tiling_on_sc=False)


@functools.partial(
    pl.kernel,
    out_type=jax.ShapeDtypeStruct((NC * NP,), jnp.float32),
    mesh=_mesh,
    compiler_params=_sc_params,
    scratch_types=[
        pltpu.VMEM((NCHUNK, K), jnp.int32),
        pltpu.VMEM((RPT,), jnp.float32),
        pltpu.VMEM((K,), jnp.float32),
        pltpu.VMEM_SHARED((NP,), jnp.float32),
    ],
)
def _deg_kernel(dst3d_hbm, degp_hbm, didx_v, z_v, ones_v, deg_sh):
    c = lax.axis_index("c")
    s = lax.axis_index("s")
    wid = c * NS + s

    def fill_z(i, _):
        z_v[pl.ds(i * 16, 16)] = jnp.zeros((16,), jnp.float32)
        return 0

    lax.fori_loop(0, RPT // 16, fill_z, 0)

    def fill_o(i, _):
        ones_v[pl.ds(i * 16, 16)] = jnp.ones((16,), jnp.float32)
        return 0

    lax.fori_loop(0, K // 16, fill_o, 0)

    pltpu.sync_copy(z_v, deg_sh.at[pl.ds(s * RPT, RPT)])
    pltpu.sync_copy(dst3d_hbm.at[wid], didx_v)
    plsc.subcore_barrier()

    def body(ci, _):
        pltpu.sync_copy(ones_v, deg_sh.at[didx_v.at[ci]], add=True)
        return 0

    lax.fori_loop(0, NCHUNK, body, 0)
    plsc.subcore_barrier()
    pltpu.sync_copy(deg_sh.at[pl.ds(s * RPT, RPT)],
                    degp_hbm.at[pl.ds(c * NP + s * RPT, RPT)])


@functools.partial(
    pl.kernel,
    out_type=jax.ShapeDtypeStruct((NC * NP, HID), jnp.float32),
    mesh=_mesh,
    compiler_params=_sc_params,
    scratch_types=[
        pltpu.VMEM((NCHUNK, K), jnp.int32),
        pltpu.VMEM((NCHUNK, K), jnp.int32),
    ] + [pltpu.VMEM((K, HID), jnp.float32) for _ in range(NBUF)] + [
        pltpu.VMEM_SHARED((NP, HID), jnp.float32),
        pltpu.VMEM_SHARED((NP, HID), jnp.float32),
        pltpu.SemaphoreType.DMA,
        pltpu.SemaphoreType.DMA,
    ],
)
def _edge_kernel(ht_hbm, src3d_hbm, dst3d_hbm, aggp_hbm,
                 sidx_v, didx_v, r0, r1, r2, r3, r4, agg_sh, ht_sh,
                 sem_g, sem_s):
    rows = (r0, r1, r2, r3, r4)
    c = lax.axis_index("c")
    s = lax.axis_index("s")
    wid = c * NS + s

    def fill_z(j, _):
        for l in range(HID // 16):
            r0[j, pl.ds(l * 16, 16)] = jnp.zeros((16,), jnp.float32)
        return 0

    lax.fori_loop(0, K, fill_z, 0)

    def zcp(j, _):
        pltpu.sync_copy(r0, agg_sh.at[pl.ds(s * RPT + j * K, K)])
        return 0

    lax.fori_loop(0, RPT // K, zcp, 0)

    pltpu.sync_copy(src3d_hbm.at[wid], sidx_v)
    pltpu.sync_copy(dst3d_hbm.at[wid], didx_v)
    pltpu.sync_copy(ht_hbm.at[pl.ds(s * RPT, RPT)],
                    ht_sh.at[pl.ds(s * RPT, RPT)])
    plsc.subcore_barrier()

    def grp(g, _):
        @pl.when(g > 0)
        def _drain():
            for b in range(NBUF):
                pltpu.make_async_copy(ht_hbm.at[sidx_v.at[0]], rows[b],
                                      sem_s).wait()

        cps = []
        for b in range(NBUF):
            ci = g * NBUF + b
            cps.append(pltpu.async_copy(
                ht_sh.at[sidx_v.at[ci]], rows[b], sem_g))
        for b in range(NBUF):
            cps[b].wait()
            pltpu.async_copy(rows[b], agg_sh.at[didx_v.at[g * NBUF + b]],
                             sem_s, add=True)
        return 0

    lax.fori_loop(0, NGRP, grp, 0)
    for b in range(NBUF):
        pltpu.make_async_copy(ht_hbm.at[sidx_v.at[0]], rows[b], sem_s).wait()
    plsc.subcore_barrier()
    pltpu.sync_copy(agg_sh.at[pl.ds(s * RPT, RPT)],
                    aggp_hbm.at[pl.ds(c * NP + s * RPT, RPT)])


BR = 640


def _mm1_body(x_ref, w_ref, o_ref):
    o_ref[...] = jnp.dot(x_ref[...], w_ref[...],
                         preferred_element_type=jnp.float32)


def _scale_body(z_ref, dp_ref, o_ref):
    deg = dp_ref[0] + dp_ref[1] + 1.0
    dis = lax.rsqrt(deg)
    o_ref[...] = z_ref[...] * dis


def _mm2_body(ap_ref, z1_ref, dp_ref, b1_ref, w2_ref, z2_ref, ht2_ref):
    deg = dp_ref[0] + dp_ref[1] + 1.0
    dis = lax.rsqrt(deg)
    h = jnp.maximum(dis * (ap_ref[0] + ap_ref[1])
                    + (dis * dis) * z1_ref[...] + b1_ref[...], 0.0)
    z2 = jnp.dot(h, w2_ref[...], preferred_element_type=jnp.float32)
    z2_ref[...] = z2
    ht2_ref[...] = z2 * dis


def _fin_body(ap_ref, z2_ref, dp_ref, b2_ref, o_ref):
    deg = dp_ref[0] + dp_ref[1] + 1.0
    dis = lax.rsqrt(deg)
    o_ref[...] = (dis * (ap_ref[0] + ap_ref[1])
                  + (dis * dis) * z2_ref[...] + b2_ref[...])


def _row_spec(d):
    return pl.BlockSpec((BR, d), lambda i: (i, 0))


def _pair_spec(d):
    return pl.BlockSpec((NC, BR, d), lambda i: (0, i, 0))


def _full_spec(a, b):
    return pl.BlockSpec((a, b), lambda i: (0, 0))


def kernel(x, edge_index, W1, b1, W2, b2):
    src3d = edge_index[0].reshape(NW, NCHUNK, K)
    dst3d = edge_index[1].reshape(NW, NCHUNK, K)
    x_p = jnp.pad(x, ((0, NP - N_NODES), (0, 0)))

    degp = _deg_kernel(dst3d)
    degp3 = degp.reshape(NC, NP, 1)

    z1 = pl.pallas_call(
        _mm1_body,
        grid=(NP // BR,),
        in_specs=[_row_spec(IN_D), _full_spec(IN_D, HID)],
        out_specs=_row_spec(HID),
        out_shape=jax.ShapeDtypeStruct((NP, HID), jnp.float32),
    )(x_p, W1)

    ht1 = pl.pallas_call(
        _scale_body,
        grid=(NP // BR,),
        in_specs=[_row_spec(HID), _pair_spec(1)],
        out_specs=_row_spec(HID),
        out_shape=jax.ShapeDtypeStruct((NP, HID), jnp.float32),
    )(z1, degp3)

    aggp1 = _edge_kernel(ht1, src3d, dst3d).reshape(NC, NP, HID)

    z2, ht2 = pl.pallas_call(
        _mm2_body,
        grid=(NP // BR,),
        in_specs=[_pair_spec(HID), _row_spec(HID), _pair_spec(1),
                  _full_spec(1, HID), _full_spec(HID, HID)],
        out_specs=[_row_spec(HID), _row_spec(HID)],
        out_shape=[jax.ShapeDtypeStruct((NP, HID), jnp.float32),
                   jax.ShapeDtypeStruct((NP, HID), jnp.float32)],
    )(aggp1, z1, degp3, b1.reshape(1, HID), W2)

    aggp2 = _edge_kernel(ht2, src3d, dst3d).reshape(NC, NP, HID)

    out = pl.pallas_call(
        _fin_body,
        grid=(NP // BR,),
        in_specs=[_pair_spec(HID), _row_spec(HID), _pair_spec(1),
                  _full_spec(1, HID)],
        out_specs=_row_spec(HID),
        out_shape=jax.ShapeDtypeStruct((NP, HID), jnp.float32),
    )(aggp2, z2, degp3, b2.reshape(1, HID))

    return out[:N_NODES]


# speedup vs baseline: 1.5482x; 1.5482x over previous
import functools

import jax
import jax.numpy as jnp
from jax import lax
from jax.experimental import pallas as pl
from jax.experimental.pallas import tpu as pltpu
from jax.experimental.pallas import tpu_sc as plsc

N_NODES = 10000
N_EDGES = 320000
IN_D = 128
HID = 64

NC = 2
NS = 16
NW = NC * NS

NP = 10240
NPH = NP // 2
RPT = NP // NS
PPT = NPH // NS
K = 128
NCHUNK = 80
EPT = NCHUNK * K
EPAD = NW * EPT
NBUF = 5
NGRP = NCHUNK // NBUF

_mesh = plsc.VectorSubcoreMesh(core_axis_name="c", subcore_axis_name="s")
_sc_params = pltpu.CompilerParams(use_tc_tiling_on_sc=False)


@functools.partial(
    pl.kernel,
    out_type=jax.ShapeDtypeStruct((NC * NPH, 2 * HID), jnp.float32),
    mesh=_mesh,
    compiler_params=_sc_params,
    scratch_types=[
        pltpu.VMEM((NCHUNK, K), jnp.int32),
        pltpu.VMEM((RPT,), jnp.float32),
        pltpu.VMEM((K,), jnp.float32),
        pltpu.VMEM((PPT, 2 * HID), jnp.float32),
        pltpu.VMEM_SHARED((NP,), jnp.float32),
    ],
)
def _deg_kernel(ei_hbm, degb_hbm, didx_v, dv, ones_v, pbuf, deg_sh):
    c = lax.axis_index("c")
    s = lax.axis_index("s")
    wid = c * NS + s

    def fill_z(i, _):
        dv[pl.ds(i * 16, 16)] = jnp.zeros((16,), jnp.float32)
        return 0

    lax.fori_loop(0, RPT // 16, fill_z, 0)

    def fill_o(i, _):
        ones_v[pl.ds(i * 16, 16)] = jnp.ones((16,), jnp.float32)
        return 0

    lax.fori_loop(0, K // 16, fill_o, 0)

    pltpu.sync_copy(dv, deg_sh.at[pl.ds(s * RPT, RPT)])
    pltpu.sync_copy(ei_hbm.at[1, wid], didx_v)
    plsc.subcore_barrier()

    def body(ci, _):
        pltpu.sync_copy(ones_v, deg_sh.at[didx_v.at[ci]], add=True)
        return 0

    lax.fori_loop(0, NCHUNK, body, 0)
    plsc.subcore_barrier()

    pltpu.sync_copy(deg_sh.at[pl.ds(s * RPT, RPT)], dv)

    def bc(j16, _):
        v = dv[pl.ds(j16 * 16, 16)]
        for l in range(16):
            row = 8 * j16 + l // 2
            col0 = (l % 2) * HID
            sp = jnp.full((16,), 1.0, jnp.float32) * v[l]
            for q in range(HID // 16):
                pbuf[row, pl.ds(col0 + q * 16, 16)] = sp
        return 0

    lax.fori_loop(0, RPT // 16, bc, 0)
    pltpu.sync_copy(pbuf, degb_hbm.at[pl.ds(c * NPH + s * PPT, PPT)])


@functools.partial(
    pl.kernel,
    out_type=jax.ShapeDtypeStruct((NC * NP, HID), jnp.float32),
    mesh=_mesh,
    compiler_params=_sc_params,
    scratch_types=[
        pltpu.VMEM((NCHUNK, K), jnp.int32),
        pltpu.VMEM((NCHUNK, K), jnp.int32),
    ] + [pltpu.VMEM((K, HID), jnp.float32) for _ in range(NBUF)] + [
        pltpu.VMEM_SHARED((NP, HID), jnp.float32),
        pltpu.SemaphoreType.DMA,
        pltpu.SemaphoreType.DMA,
    ],
)
def _edge_kernel(ht_hbm, ei_hbm, aggp_hbm,
                 sidx_v, didx_v, r0, r1, r2, r3, r4, agg_sh, sem_g, sem_s):
    rows = (r0, r1, r2, r3, r4)
    c = lax.axis_index("c")
    s = lax.axis_index("s")
    wid = c * NS + s

    def fill_z(j, _):
        for l in range(HID // 16):
            r0[j, pl.ds(l * 16, 16)] = jnp.zeros((16,), jnp.float32)
        return 0

    lax.fori_loop(0, K, fill_z, 0)

    def zcp(j, _):
        pltpu.sync_copy(r0, agg_sh.at[pl.ds(s * RPT + j * K, K)])
        return 0

    lax.fori_loop(0, RPT // K, zcp, 0)

    pltpu.sync_copy(ei_hbm.at[0, wid], sidx_v)
    pltpu.sync_copy(ei_hbm.at[1, wid], didx_v)
    plsc.subcore_barrier()

    def grp(g, _):
        @pl.when(g > 0)
        def _drain():
            for b in range(NBUF):
                pltpu.make_async_copy(ht_hbm.at[sidx_v.at[0]], rows[b],
                                      sem_s).wait()

        cps = []
        for b in range(NBUF):
            ci = g * NBUF + b
            cps.append(pltpu.async_copy(
                ht_hbm.at[sidx_v.at[ci]], rows[b], sem_g))
        for b in range(NBUF):
            cps[b].wait()
            pltpu.async_copy(rows[b], agg_sh.at[didx_v.at[g * NBUF + b]],
                             sem_s, add=True)
        return 0

    lax.fori_loop(0, NGRP, grp, 0)
    for b in range(NBUF):
        pltpu.make_async_copy(ht_hbm.at[sidx_v.at[0]], rows[b], sem_s).wait()
    plsc.subcore_barrier()
    pltpu.sync_copy(agg_sh.at[pl.ds(s * RPT, RPT)],
                    aggp_hbm.at[pl.ds(c * NP + s * RPT, RPT)])


PB = 320


def _mm1_body(x_ref, w_ref, o_ref):
    o_ref[...] = jnp.dot(x_ref[...], w_ref[...],
                         preferred_element_type=jnp.float32)


def _scale_body(z_ref, d0_ref, d1_ref, o_ref):
    dis = lax.rsqrt(d0_ref[...] + d1_ref[...] + 1.0)
    o_ref[...] = z_ref[...] * dis


def _mm2_body(a0_ref, a1_ref, z1_ref, d0_ref, d1_ref, b1_ref, w2_ref,
              z2_ref, ht2_ref):
    dis = lax.rsqrt(d0_ref[...] + d1_ref[...] + 1.0)
    h = jnp.maximum(dis * (a0_ref[...] + a1_ref[...])
                    + (dis * dis) * z1_ref[...] + b1_ref[...], 0.0)
    z2 = jnp.dot(h, w2_ref[...], preferred_element_type=jnp.float32)
    z2_ref[...] = z2
    ht2_ref[...] = z2 * dis


def _fin_body(a_ref, z2_ref, d_ref, b2_ref, o_ref):
    dis = lax.rsqrt(d_ref[0] + d_ref[1] + 1.0)
    o_ref[...] = (dis * (a_ref[0] + a_ref[1])
                  + (dis * dis) * z2_ref[...] + b2_ref[...])


def _pk(d=128, nb=PB):
    return pl.BlockSpec((nb, d), lambda i: (i, 0))


def _pk1(nb=PB):
    return pl.BlockSpec((nb, 128), lambda i: (i + NPH // nb, 0))


def _full_spec(a, b):
    return pl.BlockSpec((a, b), lambda i: (0, 0))


def _dup_w(w, d):
    wp = jnp.zeros((2 * d, 128), jnp.float32)
    wp = wp.at[:d, :HID].set(w)
    return wp.at[d:, HID:].set(w)


def kernel(x, edge_index, W1, b1, W2, b2):
    fill = (jnp.arange(EPAD - N_EDGES, dtype=jnp.int32) % (NP - N_NODES)
            + N_NODES)
    ei = jnp.concatenate(
        [edge_index, jnp.stack([fill, fill])], axis=1
    ).reshape(2, NW, NCHUNK, K)
    x_p = jnp.pad(x, ((0, NP - N_NODES), (0, 0)))
    xp = x_p.reshape(NPH, 2 * IN_D)
    W1p = _dup_w(W1, IN_D)
    W2p = _dup_w(W2, HID)
    b1p = jnp.concatenate([b1, b1]).reshape(1, 2 * HID)
    b2p = jnp.concatenate([b2, b2]).reshape(1, 2 * HID)

    degb = _deg_kernel(ei)

    z1p = pl.pallas_call(
        _mm1_body,
        grid=(NPH // PB,),
        in_specs=[_pk(2 * IN_D), _full_spec(2 * IN_D, 128)],
        out_specs=_pk(),
        out_shape=jax.ShapeDtypeStruct((NPH, 128), jnp.float32),
    )(xp, W1p)

    ht1p = pl.pallas_call(
        _scale_body,
        grid=(NPH // PB,),
        in_specs=[_pk(), _pk(), _pk1()],
        out_specs=_pk(),
        out_shape=jax.ShapeDtypeStruct((NPH, 128), jnp.float32),
    )(z1p, degb, degb)

    aggp1 = _edge_kernel(ht1p.reshape(NP, HID), ei).reshape(NC * NPH, 128)

    z2p, ht2p = pl.pallas_call(
        _mm2_body,
        grid=(NPH // PB,),
        in_specs=[_pk(), _pk1(), _pk(), _pk(), _pk1(),
                  _full_spec(1, 128), _full_spec(128, 128)],
        out_specs=[_pk(), _pk()],
        out_shape=[jax.ShapeDtypeStruct((NPH, 128), jnp.float32),
                   jax.ShapeDtypeStruct((NPH, 128), jnp.float32)],
    )(aggp1, aggp1, z1p, degb, degb, b1p, W2p)

    aggp2 = _edge_kernel(ht2p.reshape(NP, HID), ei).reshape(NC * NPH, 128)

    FB = 1000
    aggp2_3 = aggp2.reshape(NC, NPH, 128)
    degb3 = degb.reshape(NC, NPH, 128)
    outp = pl.pallas_call(
        _fin_body,
        grid=(N_NODES // (2 * FB),),
        in_specs=[pl.BlockSpec((NC, FB, 128), lambda i: (0, i, 0)),
                  pl.BlockSpec((FB, 128), lambda i: (i, 0)),
                  pl.BlockSpec((NC, FB, 128), lambda i: (0, i, 0)),
                  _full_spec(1, 128)],
        out_specs=pl.BlockSpec((FB, 128), lambda i: (i, 0)),
        out_shape=jax.ShapeDtypeStruct((N_NODES // 2, 128), jnp.float32),
    )(aggp2_3, z2p, degb3, b2p)

    return outp.reshape(N_NODES, HID)


# speedup vs baseline: 1.6762x; 1.0826x over previous
import functools

import jax
import jax.numpy as jnp
from jax import lax
from jax.experimental import pallas as pl
from jax.experimental.pallas import tpu as pltpu
from jax.experimental.pallas import tpu_sc as plsc

N_NODES = 10000
N_EDGES = 320000
IN_D = 128
HID = 64

NC = 2
NS = 16
NW = NC * NS

NP = 10240
NPH = NP // 2
RPT = NP // NS
PPT = NPH // NS
K = 128
NCHUNK = 80
EPT = NCHUNK * K
EPAD = NW * EPT
NBUF = 8
NGRP = NCHUNK // NBUF

_mesh = plsc.VectorSubcoreMesh(core_axis_name="c", subcore_axis_name="s")
_sc_params = pltpu.CompilerParams(use_tc_tiling_on_sc=False)


@functools.partial(
    pl.kernel,
    out_type=jax.ShapeDtypeStruct((NC * NPH, 2 * HID), jnp.float32),
    mesh=_mesh,
    compiler_params=_sc_params,
    scratch_types=[
        pltpu.VMEM((NCHUNK, K), jnp.int32),
        pltpu.VMEM((RPT,), jnp.float32),
        pltpu.VMEM((K,), jnp.float32),
        pltpu.VMEM((PPT, 2 * HID), jnp.float32),
        pltpu.VMEM_SHARED((NP,), jnp.float32),
    ],
)
def _deg_kernel(ei_hbm, degb_hbm, didx_v, dv, ones_v, pbuf, deg_sh):
    c = lax.axis_index("c")
    s = lax.axis_index("s")
    wid = c * NS + s

    def fill_z(i, _):
        dv[pl.ds(i * 16, 16)] = jnp.zeros((16,), jnp.float32)
        return 0

    lax.fori_loop(0, RPT // 16, fill_z, 0)

    def fill_o(i, _):
        ones_v[pl.ds(i * 16, 16)] = jnp.ones((16,), jnp.float32)
        return 0

    lax.fori_loop(0, K // 16, fill_o, 0)

    pltpu.sync_copy(dv, deg_sh.at[pl.ds(s * RPT, RPT)])
    pltpu.sync_copy(ei_hbm.at[1, wid], didx_v)
    plsc.subcore_barrier()

    def body(ci, _):
        pltpu.sync_copy(ones_v, deg_sh.at[didx_v.at[ci]], add=True)
        return 0

    lax.fori_loop(0, NCHUNK, body, 0)
    plsc.subcore_barrier()

    pltpu.sync_copy(deg_sh.at[pl.ds(s * RPT, RPT)], dv)

    def bc(j16, _):
        v = dv[pl.ds(j16 * 16, 16)]
        for l in range(16):
            row = 8 * j16 + l // 2
            col0 = (l % 2) * HID
            sp = jnp.full((16,), 1.0, jnp.float32) * v[l]
            for q in range(HID // 16):
                pbuf[row, pl.ds(col0 + q * 16, 16)] = sp
        return 0

    lax.fori_loop(0, RPT // 16, bc, 0)
    pltpu.sync_copy(pbuf, degb_hbm.at[pl.ds(c * NPH + s * PPT, PPT)])


@functools.partial(
    pl.kernel,
    out_type=jax.ShapeDtypeStruct((NC * NP, HID), jnp.float32),
    mesh=_mesh,
    compiler_params=_sc_params,
    scratch_types=[
        pltpu.VMEM((NCHUNK, K), jnp.int32),
        pltpu.VMEM((NCHUNK, K), jnp.int32),
    ] + [pltpu.VMEM((K, HID), jnp.float32) for _ in range(NBUF)] + [
        pltpu.VMEM_SHARED((NP, HID), jnp.float32),
        pltpu.SemaphoreType.DMA,
        pltpu.SemaphoreType.DMA,
    ],
)
def _edge_kernel(ht_hbm, ei_hbm, aggp_hbm,
                 sidx_v, didx_v, r0, r1, r2, r3, r4, r5, r6, r7,
                 agg_sh, sem_g, sem_s):
    rows = (r0, r1, r2, r3, r4, r5, r6, r7)
    c = lax.axis_index("c")
    s = lax.axis_index("s")
    wid = c * NS + s

    def fill_z(j, _):
        for l in range(HID // 16):
            r0[j, pl.ds(l * 16, 16)] = jnp.zeros((16,), jnp.float32)
        return 0

    lax.fori_loop(0, K, fill_z, 0)

    def zcp(j, _):
        pltpu.sync_copy(r0, agg_sh.at[pl.ds(s * RPT + j * K, K)])
        return 0

    lax.fori_loop(0, RPT // K, zcp, 0)

    pltpu.sync_copy(ei_hbm.at[0, wid], sidx_v)
    pltpu.sync_copy(ei_hbm.at[1, wid], didx_v)
    plsc.subcore_barrier()

    def grp(g, _):
        @pl.when(g > 0)
        def _drain():
            for b in range(NBUF):
                pltpu.make_async_copy(ht_hbm.at[sidx_v.at[0]], rows[b],
                                      sem_s).wait()

        cps = []
        for b in range(NBUF):
            ci = g * NBUF + b
            cps.append(pltpu.async_copy(
                ht_hbm.at[sidx_v.at[ci]], rows[b], sem_g))
        for b in range(NBUF):
            cps[b].wait()
            pltpu.async_copy(rows[b], agg_sh.at[didx_v.at[g * NBUF + b]],
                             sem_s, add=True)
        return 0

    lax.fori_loop(0, NGRP, grp, 0)
    for b in range(NBUF):
        pltpu.make_async_copy(ht_hbm.at[sidx_v.at[0]], rows[b], sem_s).wait()
    plsc.subcore_barrier()
    pltpu.sync_copy(agg_sh.at[pl.ds(s * RPT, RPT)],
                    aggp_hbm.at[pl.ds(c * NP + s * RPT, RPT)])


PB = 640


def _mm1_body(x_ref, w_ref, d0_ref, d1_ref, z_ref, ht_ref):
    dis = lax.rsqrt(d0_ref[...] + d1_ref[...] + 1.0)
    z = jnp.dot(x_ref[...], w_ref[...], preferred_element_type=jnp.float32)
    z_ref[...] = z
    ht_ref[...] = z * dis


def _mm2_body(a0_ref, a1_ref, z1_ref, d0_ref, d1_ref, b1_ref, w2_ref,
              z2_ref, ht2_ref):
    dis = lax.rsqrt(d0_ref[...] + d1_ref[...] + 1.0)
    h = jnp.maximum(dis * (a0_ref[...] + a1_ref[...])
                    + (dis * dis) * z1_ref[...] + b1_ref[...], 0.0)
    z2 = jnp.dot(h, w2_ref[...], preferred_element_type=jnp.float32)
    z2_ref[...] = z2
    ht2_ref[...] = z2 * dis


def _fin_body(a_ref, z2_ref, d_ref, b2_ref, o_ref):
    dis = lax.rsqrt(d_ref[0] + d_ref[1] + 1.0)
    o_ref[...] = (dis * (a_ref[0] + a_ref[1])
                  + (dis * dis) * z2_ref[...] + b2_ref[...])


def _pk(d=128, nb=PB):
    return pl.BlockSpec((nb, d), lambda i: (i, 0))


def _pk1(nb=PB):
    return pl.BlockSpec((nb, 128), lambda i: (i + NPH // nb, 0))


def _full_spec(a, b):
    return pl.BlockSpec((a, b), lambda i: (0, 0))


def _dup_w(w, d):
    wp = jnp.zeros((2 * d, 128), jnp.float32)
    wp = wp.at[:d, :HID].set(w)
    return wp.at[d:, HID:].set(w)


def kernel(x, edge_index, W1, b1, W2, b2):
    fill = (jnp.arange(EPAD - N_EDGES, dtype=jnp.int32) % (NP - N_NODES)
            + N_NODES)
    ei = jnp.concatenate(
        [edge_index, jnp.stack([fill, fill])], axis=1
    ).reshape(2, NW, NCHUNK, K)
    x_p = jnp.pad(x, ((0, NP - N_NODES), (0, 0)))
    xp = x_p.reshape(NPH, 2 * IN_D)
    W1p = _dup_w(W1, IN_D)
    W2p = _dup_w(W2, HID)
    b1p = jnp.concatenate([b1, b1]).reshape(1, 2 * HID)
    b2p = jnp.concatenate([b2, b2]).reshape(1, 2 * HID)

    degb = _deg_kernel(ei)

    z1p, ht1p = pl.pallas_call(
        _mm1_body,
        grid=(NPH // PB,),
        in_specs=[_pk(2 * IN_D), _full_spec(2 * IN_D, 128), _pk(), _pk1()],
        out_specs=[_pk(), _pk()],
        out_shape=[jax.ShapeDtypeStruct((NPH, 128), jnp.float32),
                   jax.ShapeDtypeStruct((NPH, 128), jnp.float32)],
    )(xp, W1p, degb, degb)

    aggp1 = _edge_kernel(ht1p.reshape(NP, HID), ei).reshape(NC * NPH, 128)

    z2p, ht2p = pl.pallas_call(
        _mm2_body,
        grid=(NPH // PB,),
        in_specs=[_pk(), _pk1(), _pk(), _pk(), _pk1(),
                  _full_spec(1, 128), _full_spec(128, 128)],
        out_specs=[_pk(), _pk()],
        out_shape=[jax.ShapeDtypeStruct((NPH, 128), jnp.float32),
                   jax.ShapeDtypeStruct((NPH, 128), jnp.float32)],
    )(aggp1, aggp1, z1p, degb, degb, b1p, W2p)

    aggp2 = _edge_kernel(ht2p.reshape(NP, HID), ei).reshape(NC * NPH, 128)

    FB = 1000
    aggp2_3 = aggp2.reshape(NC, NPH, 128)
    degb3 = degb.reshape(NC, NPH, 128)
    outp = pl.pallas_call(
        _fin_body,
        grid=(N_NODES // (2 * FB),),
        in_specs=[pl.BlockSpec((NC, FB, 128), lambda i: (0, i, 0)),
                  pl.BlockSpec((FB, 128), lambda i: (i, 0)),
                  pl.BlockSpec((NC, FB, 128), lambda i: (0, i, 0)),
                  _full_spec(1, 128)],
        out_specs=pl.BlockSpec((FB, 128), lambda i: (i, 0)),
        out_shape=jax.ShapeDtypeStruct((N_NODES // 2, 128), jnp.float32),
    )(aggp2_3, z2p, degb3, b2p)

    return outp.reshape(N_NODES, HID)


# speedup vs baseline: 1.6776x; 1.0009x over previous
import functools

import jax
import jax.numpy as jnp
from jax import lax
from jax.experimental import pallas as pl
from jax.experimental.pallas import tpu as pltpu
from jax.experimental.pallas import tpu_sc as plsc

N_NODES = 10000
N_EDGES = 320000
IN_D = 128
HID = 64

NC = 2
NS = 16
NW = NC * NS

NP = 10240
NPH = NP // 2
RPT = NP // NS
PPT = NPH // NS
K = 128
NCHUNK = 80
EPT = NCHUNK * K
EPAD = NW * EPT
NBUF = 8
NGRP = NCHUNK // NBUF

_mesh = plsc.VectorSubcoreMesh(core_axis_name="c", subcore_axis_name="s")
_sc_params = pltpu.CompilerParams(use_tc_tiling_on_sc=False)


@functools.partial(
    pl.kernel,
    out_type=jax.ShapeDtypeStruct((NC * NPH, 2 * HID), jnp.float32),
    mesh=_mesh,
    compiler_params=_sc_params,
    scratch_types=[
        pltpu.VMEM((NCHUNK, K), jnp.int32),
        pltpu.VMEM((RPT,), jnp.float32),
        pltpu.VMEM((K,), jnp.float32),
        pltpu.VMEM((PPT, 2 * HID), jnp.float32),
        pltpu.VMEM_SHARED((NP,), jnp.float32),
    ],
)
def _deg_kernel(ei_hbm, degb_hbm, didx_v, dv, ones_v, pbuf, deg_sh):
    c = lax.axis_index("c")
    s = lax.axis_index("s")
    wid = c * NS + s

    def fill_z(i, _):
        dv[pl.ds(i * 16, 16)] = jnp.zeros((16,), jnp.float32)
        return 0

    lax.fori_loop(0, RPT // 16, fill_z, 0)

    def fill_o(i, _):
        ones_v[pl.ds(i * 16, 16)] = jnp.ones((16,), jnp.float32)
        return 0

    lax.fori_loop(0, K // 16, fill_o, 0)

    pltpu.sync_copy(dv, deg_sh.at[pl.ds(s * RPT, RPT)])
    pltpu.sync_copy(ei_hbm.at[1, wid], didx_v)
    plsc.subcore_barrier()

    def body(ci, _):
        pltpu.sync_copy(ones_v, deg_sh.at[didx_v.at[ci]], add=True)
        return 0

    lax.fori_loop(0, NCHUNK, body, 0)
    plsc.subcore_barrier()

    pltpu.sync_copy(deg_sh.at[pl.ds(s * RPT, RPT)], dv)

    def bc(j16, _):
        v = dv[pl.ds(j16 * 16, 16)]
        for l in range(16):
            row = 8 * j16 + l // 2
            col0 = (l % 2) * HID
            sp = jnp.full((16,), 1.0, jnp.float32) * v[l]
            for q in range(HID // 16):
                pbuf[row, pl.ds(col0 + q * 16, 16)] = sp
        return 0

    lax.fori_loop(0, RPT // 16, bc, 0)
    pltpu.sync_copy(pbuf, degb_hbm.at[pl.ds(c * NPH + s * PPT, PPT)])


@functools.partial(
    pl.kernel,
    out_type=jax.ShapeDtypeStruct((NC * NP, HID), jnp.float32),
    mesh=_mesh,
    compiler_params=_sc_params,
    scratch_types=[
        pltpu.VMEM((NCHUNK, K), jnp.int32),
        pltpu.VMEM((NCHUNK, K), jnp.int32),
    ] + [pltpu.VMEM((K, HID), jnp.float32) for _ in range(NBUF)] + [
        pltpu.VMEM_SHARED((NP, HID), jnp.float32),
        pltpu.SemaphoreType.DMA,
        pltpu.SemaphoreType.DMA,
    ],
)
def _edge_kernel(ht_hbm, ei_hbm, aggp_hbm,
                 sidx_v, didx_v, r0, r1, r2, r3, r4, r5, r6, r7,
                 agg_sh, sem_g, sem_s):
    rows = (r0, r1, r2, r3, r4, r5, r6, r7)
    c = lax.axis_index("c")
    s = lax.axis_index("s")
    wid = c * NS + s

    def fill_z(j, _):
        for l in range(HID // 16):
            r0[j, pl.ds(l * 16, 16)] = jnp.zeros((16,), jnp.float32)
        return 0

    lax.fori_loop(0, K, fill_z, 0)

    def zcp(j, _):
        pltpu.sync_copy(r0, agg_sh.at[pl.ds(s * RPT + j * K, K)])
        return 0

    lax.fori_loop(0, RPT // K, zcp, 0)

    pltpu.sync_copy(ei_hbm.at[0, wid], sidx_v)
    pltpu.sync_copy(ei_hbm.at[1, wid], didx_v)
    plsc.subcore_barrier()

    def grp(g, _):
        @pl.when(g > 0)
        def _drain():
            for b in range(NBUF):
                pltpu.make_async_copy(ht_hbm.at[sidx_v.at[0]], rows[b],
                                      sem_s).wait()

        cps = []
        for b in range(NBUF):
            ci = g * NBUF + b
            cps.append(pltpu.async_copy(
                ht_hbm.at[sidx_v.at[ci]], rows[b], sem_g))
        for b in range(NBUF):
            cps[b].wait()
            pltpu.async_copy(rows[b], agg_sh.at[didx_v.at[g * NBUF + b]],
                             sem_s, add=True)
        return 0

    lax.fori_loop(0, NGRP, grp, 0)
    for b in range(NBUF):
        pltpu.make_async_copy(ht_hbm.at[sidx_v.at[0]], rows[b], sem_s).wait()
    plsc.subcore_barrier()
    pltpu.sync_copy(agg_sh.at[pl.ds(s * RPT, RPT)],
                    aggp_hbm.at[pl.ds(c * NP + s * RPT, RPT)])


PB = 640


def _mm1_body(x_ref, w_ref, d0_ref, d1_ref, z_ref, ht_ref, dis_ref):
    dis = lax.rsqrt(d0_ref[...] + d1_ref[...] + 1.0)
    z = jnp.dot(x_ref[...], w_ref[...], preferred_element_type=jnp.float32)
    z_ref[...] = z
    ht_ref[...] = z * dis
    dis_ref[...] = dis


def _mm2_body(a0_ref, a1_ref, z1_ref, dis_ref, b1_ref, w2_ref,
              z2_ref, ht2_ref):
    dis = dis_ref[...]
    h = jnp.maximum(dis * (a0_ref[...] + a1_ref[...])
                    + (dis * dis) * z1_ref[...] + b1_ref[...], 0.0)
    z2 = jnp.dot(h, w2_ref[...], preferred_element_type=jnp.float32)
    z2_ref[...] = z2
    ht2_ref[...] = z2 * dis


def _fin_body(a_ref, z2_ref, dis_ref, b2_ref, o_ref):
    dis = dis_ref[...]
    o_ref[...] = (dis * (a_ref[0] + a_ref[1])
                  + (dis * dis) * z2_ref[...] + b2_ref[...])


def _pk(d=128, nb=PB):
    return pl.BlockSpec((nb, d), lambda i: (i, 0))


def _pk1(nb=PB):
    return pl.BlockSpec((nb, 128), lambda i: (i + NPH // nb, 0))


def _full_spec(a, b):
    return pl.BlockSpec((a, b), lambda i: (0, 0))


def _dup_w(w, d):
    wp = jnp.zeros((2 * d, 128), jnp.float32)
    wp = wp.at[:d, :HID].set(w)
    return wp.at[d:, HID:].set(w)


def kernel(x, edge_index, W1, b1, W2, b2):
    fill = (jnp.arange(EPAD - N_EDGES, dtype=jnp.int32) % (NP - N_NODES)
            + N_NODES)
    ei = jnp.concatenate(
        [edge_index, jnp.stack([fill, fill])], axis=1
    ).reshape(2, NW, NCHUNK, K)
    x_p = jnp.pad(x, ((0, NP - N_NODES), (0, 0)))
    xp = x_p.reshape(NPH, 2 * IN_D)
    W1p = _dup_w(W1, IN_D)
    W2p = _dup_w(W2, HID)
    b1p = jnp.concatenate([b1, b1]).reshape(1, 2 * HID)
    b2p = jnp.concatenate([b2, b2]).reshape(1, 2 * HID)

    degb = _deg_kernel(ei)

    z1p, ht1p, disp = pl.pallas_call(
        _mm1_body,
        grid=(NPH // PB,),
        in_specs=[_pk(2 * IN_D), _full_spec(2 * IN_D, 128), _pk(), _pk1()],
        out_specs=[_pk(), _pk(), _pk()],
        out_shape=[jax.ShapeDtypeStruct((NPH, 128), jnp.float32),
                   jax.ShapeDtypeStruct((NPH, 128), jnp.float32),
                   jax.ShapeDtypeStruct((NPH, 128), jnp.float32)],
    )(xp, W1p, degb, degb)

    aggp1 = _edge_kernel(ht1p.reshape(NP, HID), ei).reshape(NC * NPH, 128)

    z2p, ht2p = pl.pallas_call(
        _mm2_body,
        grid=(NPH // PB,),
        in_specs=[_pk(), _pk1(), _pk(), _pk(),
                  _full_spec(1, 128), _full_spec(128, 128)],
        out_specs=[_pk(), _pk()],
        out_shape=[jax.ShapeDtypeStruct((NPH, 128), jnp.float32),
                   jax.ShapeDtypeStruct((NPH, 128), jnp.float32)],
    )(aggp1, aggp1, z1p, disp, b1p, W2p)

    aggp2 = _edge_kernel(ht2p.reshape(NP, HID), ei).reshape(NC * NPH, 128)

    FB = 1000
    aggp2_3 = aggp2.reshape(NC, NPH, 128)
    out = pl.pallas_call(
        _fin_body,
        grid=(N_NODES // (2 * FB),),
        in_specs=[pl.BlockSpec((NC, FB, 128), lambda i: (0, i, 0)),
                  pl.BlockSpec((FB, 128), lambda i: (i, 0)),
                  pl.BlockSpec((FB, 128), lambda i: (i, 0)),
                  _full_spec(1, 128)],
        out_specs=pl.BlockSpec((FB, 128), lambda i: (i, 0)),
        out_shape=jax.ShapeDtypeStruct((N_NODES // 2, 128), jnp.float32),
    )(aggp2_3, z2p, disp, b2p)

    return out.reshape(N_NODES, HID)


# speedup vs baseline: 1.7253x; 1.0284x over previous
import functools

import jax
import jax.numpy as jnp
from jax import lax
from jax.experimental import pallas as pl
from jax.experimental.pallas import tpu as pltpu
from jax.experimental.pallas import tpu_sc as plsc

N_NODES = 10000
N_EDGES = 320000
IN_D = 128
HID = 64

NC = 2
NS = 16
NW = NC * NS

NP = 10240
NPH = NP // 2
RPT = NP // NS
PPT = NPH // NS
K = 128
NCHUNK = 80
EPT = NCHUNK * K
EPAD = NW * EPT
NBUF = 8
NGRP = NCHUNK // NBUF

_mesh = plsc.VectorSubcoreMesh(core_axis_name="c", subcore_axis_name="s")
_sc_params = pltpu.CompilerParams(use_tc_tiling_on_sc=False)


@functools.partial(
    pl.kernel,
    out_type=jax.ShapeDtypeStruct((NC * NPH, 2 * HID), jnp.float32),
    mesh=_mesh,
    compiler_params=_sc_params,
    scratch_types=[
        pltpu.VMEM((NCHUNK, K), jnp.int32),
        pltpu.VMEM((RPT,), jnp.float32),
        pltpu.VMEM((K,), jnp.float32),
        pltpu.VMEM((PPT, 2 * HID), jnp.float32),
        pltpu.VMEM_SHARED((NP,), jnp.float32),
        pltpu.SemaphoreType.DMA,
    ],
)
def _deg_kernel(ei_hbm, degb_hbm, didx_v, dv, ones_v, pbuf, deg_sh, sem):
    c = lax.axis_index("c")
    s = lax.axis_index("s")
    wid = c * NS + s

    def fill_z(i, _):
        dv[pl.ds(i * 16, 16)] = jnp.zeros((16,), jnp.float32)
        return 0

    lax.fori_loop(0, RPT // 16, fill_z, 0)

    def fill_o(i, _):
        ones_v[pl.ds(i * 16, 16)] = jnp.ones((16,), jnp.float32)
        return 0

    lax.fori_loop(0, K // 16, fill_o, 0)

    pltpu.sync_copy(dv, deg_sh.at[pl.ds(s * RPT, RPT)])
    pltpu.sync_copy(ei_hbm.at[1, wid], didx_v)
    plsc.subcore_barrier()

    def body(ci, _):
        pltpu.async_copy(ones_v, deg_sh.at[didx_v.at[ci]], sem, add=True)
        return 0

    lax.fori_loop(0, NCHUNK, body, 0)

    def drain(ci, _):
        pltpu.make_async_copy(ei_hbm.at[1, wid, ci], didx_v.at[0], sem).wait()
        return 0

    lax.fori_loop(0, NCHUNK, drain, 0)
    plsc.subcore_barrier()

    pltpu.sync_copy(deg_sh.at[pl.ds(s * RPT, RPT)], dv)

    def bc(j16, _):
        v = dv[pl.ds(j16 * 16, 16)]
        for l in range(16):
            row = 8 * j16 + l // 2
            col0 = (l % 2) * HID
            sp = jnp.full((16,), 1.0, jnp.float32) * v[l]
            for q in range(HID // 16):
                pbuf[row, pl.ds(col0 + q * 16, 16)] = sp
        return 0

    lax.fori_loop(0, RPT // 16, bc, 0)
    pltpu.sync_copy(pbuf, degb_hbm.at[pl.ds(c * NPH + s * PPT, PPT)])


@functools.partial(
    pl.kernel,
    out_type=jax.ShapeDtypeStruct((NC * NP, HID), jnp.float32),
    mesh=_mesh,
    compiler_params=_sc_params,
    scratch_types=[
        pltpu.VMEM((NCHUNK, K), jnp.int32),
        pltpu.VMEM((NCHUNK, K), jnp.int32),
    ] + [pltpu.VMEM((K, HID), jnp.float32) for _ in range(NBUF)] + [
        pltpu.VMEM_SHARED((NP, HID), jnp.float32),
        pltpu.SemaphoreType.DMA,
        pltpu.SemaphoreType.DMA,
    ],
)
def _edge_kernel(ht_hbm, ei_hbm, aggp_hbm,
                 sidx_v, didx_v, r0, r1, r2, r3, r4, r5, r6, r7,
                 agg_sh, sem_g, sem_s):
    rows = (r0, r1, r2, r3, r4, r5, r6, r7)
    c = lax.axis_index("c")
    s = lax.axis_index("s")
    wid = c * NS + s

    def fill_z(j, _):
        for l in range(HID // 16):
            r0[j, pl.ds(l * 16, 16)] = jnp.zeros((16,), jnp.float32)
        return 0

    lax.fori_loop(0, K, fill_z, 0)

    def zcp(j, _):
        pltpu.sync_copy(r0, agg_sh.at[pl.ds(s * RPT + j * K, K)])
        return 0

    lax.fori_loop(0, RPT // K, zcp, 0)

    pltpu.sync_copy(ei_hbm.at[0, wid], sidx_v)
    pltpu.sync_copy(ei_hbm.at[1, wid], didx_v)
    plsc.subcore_barrier()

    def grp(g, _):
        @pl.when(g > 0)
        def _drain():
            for b in range(NBUF):
                pltpu.make_async_copy(ht_hbm.at[sidx_v.at[0]], rows[b],
                                      sem_s).wait()

        cps = []
        for b in range(NBUF):
            ci = g * NBUF + b
            cps.append(pltpu.async_copy(
                ht_hbm.at[sidx_v.at[ci]], rows[b], sem_g))
        for b in range(NBUF):
            cps[b].wait()
            pltpu.async_copy(rows[b], agg_sh.at[didx_v.at[g * NBUF + b]],
                             sem_s, add=True)
        return 0

    lax.fori_loop(0, NGRP, grp, 0)
    for b in range(NBUF):
        pltpu.make_async_copy(ht_hbm.at[sidx_v.at[0]], rows[b], sem_s).wait()
    plsc.subcore_barrier()
    pltpu.sync_copy(agg_sh.at[pl.ds(s * RPT, RPT)],
                    aggp_hbm.at[pl.ds(c * NP + s * RPT, RPT)])


PB = 640


def _mm1_body(x_ref, w_ref, d0_ref, d1_ref, z_ref, ht_ref, dis_ref):
    dis = lax.rsqrt(d0_ref[...] + d1_ref[...] + 1.0)
    z = jnp.dot(x_ref[...], w_ref[...], preferred_element_type=jnp.float32)
    z_ref[...] = z
    ht_ref[...] = z * dis
    dis_ref[...] = dis


def _mm2_body(a0_ref, a1_ref, z1_ref, dis_ref, b1_ref, w2_ref,
              z2_ref, ht2_ref):
    dis = dis_ref[...]
    h = jnp.maximum(dis * (a0_ref[...] + a1_ref[...])
                    + (dis * dis) * z1_ref[...] + b1_ref[...], 0.0)
    z2 = jnp.dot(h, w2_ref[...], preferred_element_type=jnp.float32)
    z2_ref[...] = z2
    ht2_ref[...] = z2 * dis


def _fin_body(a_ref, z2_ref, dis_ref, b2_ref, o_ref):
    dis = dis_ref[...]
    o_ref[...] = (dis * (a_ref[0] + a_ref[1])
                  + (dis * dis) * z2_ref[...] + b2_ref[...])


def _pk(d=128, nb=PB):
    return pl.BlockSpec((nb, d), lambda i: (i, 0))


def _pk1(nb=PB):
    return pl.BlockSpec((nb, 128), lambda i: (i + NPH // nb, 0))


def _full_spec(a, b):
    return pl.BlockSpec((a, b), lambda i: (0, 0))


def _dup_w(w, d):
    wp = jnp.zeros((2 * d, 128), jnp.float32)
    wp = wp.at[:d, :HID].set(w)
    return wp.at[d:, HID:].set(w)


def kernel(x, edge_index, W1, b1, W2, b2):
    fill = (jnp.arange(EPAD - N_EDGES, dtype=jnp.int32) % (NP - N_NODES)
            + N_NODES)
    ei = jnp.concatenate(
        [edge_index, jnp.stack([fill, fill])], axis=1
    ).reshape(2, NW, NCHUNK, K)
    x_p = jnp.pad(x, ((0, NP - N_NODES), (0, 0)))
    xp = x_p.reshape(NPH, 2 * IN_D)
    W1p = _dup_w(W1, IN_D)
    W2p = _dup_w(W2, HID)
    b1p = jnp.concatenate([b1, b1]).reshape(1, 2 * HID)
    b2p = jnp.concatenate([b2, b2]).reshape(1, 2 * HID)

    degb = _deg_kernel(ei)

    z1p, ht1p, disp = pl.pallas_call(
        _mm1_body,
        grid=(NPH // PB,),
        in_specs=[_pk(2 * IN_D), _full_spec(2 * IN_D, 128), _pk(), _pk1()],
        out_specs=[_pk(), _pk(), _pk()],
        out_shape=[jax.ShapeDtypeStruct((NPH, 128), jnp.float32),
                   jax.ShapeDtypeStruct((NPH, 128), jnp.float32),
                   jax.ShapeDtypeStruct((NPH, 128), jnp.float32)],
    )(xp, W1p, degb, degb)

    aggp1 = _edge_kernel(ht1p.reshape(NP, HID), ei).reshape(NC * NPH, 128)

    z2p, ht2p = pl.pallas_call(
        _mm2_body,
        grid=(NPH // PB,),
        in_specs=[_pk(), _pk1(), _pk(), _pk(),
                  _full_spec(1, 128), _full_spec(128, 128)],
        out_specs=[_pk(), _pk()],
        out_shape=[jax.ShapeDtypeStruct((NPH, 128), jnp.float32),
                   jax.ShapeDtypeStruct((NPH, 128), jnp.float32)],
    )(aggp1, aggp1, z1p, disp, b1p, W2p)

    aggp2 = _edge_kernel(ht2p.reshape(NP, HID), ei).reshape(NC * NPH, 128)

    FB = 1000
    aggp2_3 = aggp2.reshape(NC, NPH, 128)
    out = pl.pallas_call(
        _fin_body,
        grid=(N_NODES // (2 * FB),),
        in_specs=[pl.BlockSpec((NC, FB, 128), lambda i: (0, i, 0)),
                  pl.BlockSpec((FB, 128), lambda i: (i, 0)),
                  pl.BlockSpec((FB, 128), lambda i: (i, 0)),
                  _full_spec(1, 128)],
        out_specs=pl.BlockSpec((FB, 128), lambda i: (i, 0)),
        out_shape=jax.ShapeDtypeStruct((N_NODES // 2, 128), jnp.float32),
    )(aggp2_3, z2p, disp, b2p)

    return out.reshape(N_NODES, HID)
